# baseline (speedup 1.0000x reference)
import jax
import jax.numpy as jnp
from jax import lax
from jax.experimental import pallas as pl
from jax.experimental.pallas import tpu as pltpu
from jax.experimental.pallas import tpu_sc as plsc

N = 10000
E = 160000
D = 256
H = 128
EPS = 1e-5

NC = 2
NS = 16

DEG_K = 40
DEG_EPT = E // (NC * NS)
DEG_NCH = DEG_EPT // DEG_K

AGG_K = 80
AGG_EPT = E // NS
AGG_NCH = AGG_EPT // AGG_K

OUT_K = 80
OUT_NCH = N // OUT_K
OUT_PER_TILE = -(-OUT_NCH // NS)

_MESH = plsc.VectorSubcoreMesh(core_axis_name="c", subcore_axis_name="s")


def _deg_body(dst, z128, ones128, cnt, didx, onesv, acc,
              sem_i, sem_s):
    c = lax.axis_index("c")
    s = lax.axis_index("s")
    base = (c * NS + s) * DEG_EPT

    def iload(i, carry):
        pltpu.async_copy(dst.at[pl.ds(base + i * DEG_K, DEG_K)],
                         didx.at[i], sem_i)
        return carry

    lax.fori_loop(0, DEG_NCH, iload, 0)

    pltpu.sync_copy(ones128, onesv)

    def zbody(j, carry):
        g = s + j * NS

        @pl.when(g < OUT_NCH)
        def _():
            pltpu.sync_copy(z128, acc.at[pl.ds(g * OUT_K, OUT_K)])

        return carry

    lax.fori_loop(0, OUT_PER_TILE, zbody, 0)

    def idrain(i, carry):
        pltpu.make_async_copy(dst.at[pl.ds(base + i * DEG_K, DEG_K)],
                              didx.at[i], sem_i).wait()
        return carry

    lax.fori_loop(0, DEG_NCH, idrain, 0)
    plsc.subcore_barrier()

    def cbody(i, carry):
        pltpu.async_copy(onesv, acc.at[didx.at[i]], sem_s, add=True)
        return carry

    lax.fori_loop(0, DEG_NCH, cbody, 0)

    def cdrain(i, carry):
        pltpu.make_async_copy(onesv, acc.at[didx.at[i]], sem_s).wait()
        return carry

    lax.fori_loop(0, DEG_NCH, cdrain, 0)
    plsc.subcore_barrier()

    def obody(j, carry):
        g = s + j * NS

        @pl.when(g < OUT_NCH)
        def _():
            r = g * OUT_K
            pltpu.sync_copy(acc.at[pl.ds(r, OUT_K)], cnt.at[c, pl.ds(r, OUT_K)])

        return carry

    lax.fori_loop(0, OUT_PER_TILE, obody, 0)


_deg_call = pl.kernel(
    _deg_body,
    out_type=jax.ShapeDtypeStruct((NC, N, H), jnp.float32),
    mesh=_MESH,
    scratch_types=[
        pltpu.VMEM((DEG_NCH, DEG_K), jnp.int32),
        pltpu.VMEM((DEG_K, H), jnp.float32),
        pltpu.VMEM_SHARED((N, H), jnp.float32),
        pltpu.SemaphoreType.DMA,
        pltpu.SemaphoreType.DMA,
    ],
    name="sc_deg",
)


def _agg_body(hs2n, srccat, dst, z128, out, sidx, didx, gbuf, acc,
              sem_i, sem_g0, sem_g1):
    c = lax.axis_index("c")
    s = lax.axis_index("s")
    base = s * AGG_EPT

    pltpu.async_copy(srccat.at[pl.ds(c * E + base, AGG_EPT)], sidx, sem_i)

    def iload(i, carry):
        pltpu.async_copy(dst.at[pl.ds(base + i * AGG_K, AGG_K)],
                         didx.at[i], sem_i)
        return carry

    lax.fori_loop(0, AGG_NCH, iload, 0)

    def zbody(j, carry):
        g = s + j * NS

        @pl.when(g < OUT_NCH)
        def _():
            pltpu.sync_copy(z128, acc.at[pl.ds(g * OUT_K, OUT_K)])

        return carry

    lax.fori_loop(0, OUT_PER_TILE, zbody, 0)

    pltpu.make_async_copy(srccat.at[pl.ds(c * E + base, AGG_EPT)],
                          sidx, sem_i).wait()

    def idrain(i, carry):
        pltpu.make_async_copy(dst.at[pl.ds(base + i * AGG_K, AGG_K)],
                              didx.at[i], sem_i).wait()
        return carry

    lax.fori_loop(0, AGG_NCH, idrain, 0)
    plsc.subcore_barrier()

    def gissue(k, buf, sem):
        pltpu.async_copy(hs2n.at[sidx.at[pl.ds(k * AGG_K, AGG_K)]],
                         gbuf.at[buf], sem)

    def gwait(k, buf, sem):
        pltpu.make_async_copy(hs2n.at[sidx.at[pl.ds(k * AGG_K, AGG_K)]],
                              gbuf.at[buf], sem).wait()

    def scat(k, buf):
        pltpu.sync_copy(gbuf.at[buf], acc.at[didx.at[k]], add=True)

    gissue(0, 0, sem_g0)

    def pair(i, carry):
        k = 2 * i
        gissue(k + 1, 1, sem_g1)
        gwait(k, 0, sem_g0)
        scat(k, 0)
        gissue(k + 2, 0, sem_g0)
        gwait(k + 1, 1, sem_g1)
        scat(k + 1, 1)
        return carry

    lax.fori_loop(0, (AGG_NCH - 1) // 2, pair, 0)
    gwait(AGG_NCH - 1, 0, sem_g0)
    scat(AGG_NCH - 1, 0)
    plsc.subcore_barrier()

    def obody(j, carry):
        g = s + j * NS

        @pl.when(g < OUT_NCH)
        def _():
            r = g * OUT_K
            pltpu.sync_copy(acc.at[pl.ds(r, OUT_K)], out.at[c, pl.ds(r, OUT_K)])

        return carry

    lax.fori_loop(0, OUT_PER_TILE, obody, 0)


_agg_call = pl.kernel(
    _agg_body,
    out_type=jax.ShapeDtypeStruct((NC, N, H), jnp.float32),
    mesh=_MESH,
    scratch_types=[
        pltpu.VMEM((AGG_EPT,), jnp.int32),
        pltpu.VMEM((AGG_NCH, AGG_K), jnp.int32),
        pltpu.VMEM((2, AGG_K, H), jnp.float32),
        pltpu.VMEM_SHARED((N, H), jnp.float32),
        pltpu.SemaphoreType.DMA,
        pltpu.SemaphoreType.DMA,
        pltpu.SemaphoreType.DMA,
    ],
    name="sc_agg",
)


RB = 1000
GRID = N // RB


def _dinv_from_cnt(cnt_ref):
    deg = cnt_ref[0, :, 0:1] + cnt_ref[1, :, 0:1] + 1.0
    return lax.rsqrt(deg)


def _dinv_from_d8(d8_ref):
    return d8_ref[:, 0:1]


def _layer_norm_tc(xb, g, b):
    mu = jnp.mean(xb, axis=1, keepdims=True)
    xc = xb - mu
    var = jnp.mean(xc * xc, axis=1, keepdims=True)
    return xc * lax.rsqrt(var + EPS) * g + b


def _tc1_body(x_ref, cnt_ref, g_ref, b_ref, w_ref, hs_ref, d8_ref):
    ln = _layer_norm_tc(x_ref[...], g_ref[...], b_ref[...])
    h = jnp.dot(ln, w_ref[...], preferred_element_type=jnp.float32)
    dinv = _dinv_from_cnt(cnt_ref)
    hs = h * dinv
    hs_ref[0] = hs[:, :H]
    hs_ref[1] = hs[:, H:]
    d8_ref[...] = jnp.broadcast_to(dinv, (RB, 8))


_row_spec = pl.BlockSpec((RB, D), lambda i: (i, 0))
_pair_spec = pl.BlockSpec((NC, RB, H), lambda i: (0, i, 0))
_cnt_spec = pl.BlockSpec((NC, RB, H), lambda i: (0, i, 0))
_d8_spec = pl.BlockSpec((RB, 8), lambda i: (i, 0))
_vecD_spec = pl.BlockSpec((1, D), lambda i: (0, 0))
_vecH_spec = pl.BlockSpec((1, H), lambda i: (0, 0))
_mat_spec = pl.BlockSpec((D, D), lambda i: (0, 0))
_proj_spec = pl.BlockSpec((H, D), lambda i: (0, 0))

_tc1 = pl.pallas_call(
    _tc1_body,
    grid=(GRID,),
    in_specs=[_row_spec, _cnt_spec, _vecD_spec, _vecD_spec, _mat_spec],
    out_specs=[_pair_spec, _d8_spec],
    out_shape=[
        jax.ShapeDtypeStruct((NC, N, H), jnp.float32),
        jax.ShapeDtypeStruct((N, 8), jnp.float32),
    ],
)


def _tc2_body(a, hsp, d8, t, ce, wt, bt, wc, bc, b1, g2, be2, w2, o):
    dinv = _dinv_from_d8(d8)
    u0 = (a[0] + hsp[0]) * dinv
    u1 = (a[1] + hsp[1]) * dinv
    u = jnp.concatenate([u0, u1], axis=1)
    bias = (b1[...] + bt[...] + bc[...]
            + jnp.dot(t[...], wt[...], preferred_element_type=jnp.float32)
            + jnp.dot(ce[...], wc[...], preferred_element_type=jnp.float32))
    v = u + bias
    v = v * jax.nn.sigmoid(v)
    ln = _layer_norm_tc(v, g2[...], be2[...])
    h2 = jnp.dot(ln, w2[...], preferred_element_type=jnp.float32)
    hs2 = h2 * dinv
    o[0] = hs2[:, :H]
    o[1] = hs2[:, H:]


_tc2 = pl.pallas_call(
    _tc2_body,
    grid=(GRID,),
    in_specs=[_pair_spec, _pair_spec, _d8_spec,
              _vecH_spec, _vecH_spec, _proj_spec, _vecD_spec, _proj_spec,
              _vecD_spec, _vecD_spec, _vecD_spec, _vecD_spec, _mat_spec],
    out_specs=_pair_spec,
    out_shape=jax.ShapeDtypeStruct((NC, N, H), jnp.float32),
)


def _tc3_body(a, hsp, d8, x, t, ce, wt, bt, wc, bc, b2, o):
    dinv = _dinv_from_d8(d8)
    u0 = (a[0] + hsp[0]) * dinv
    u1 = (a[1] + hsp[1]) * dinv
    u = jnp.concatenate([u0, u1], axis=1)
    bias = (b2[...] + bt[...] + bc[...]
            + jnp.dot(t[...], wt[...], preferred_element_type=jnp.float32)
            + jnp.dot(ce[...], wc[...], preferred_element_type=jnp.float32))
    v = u + bias
    v = v * jax.nn.sigmoid(v)
    o[...] = v + x[...]


_tc3 = pl.pallas_call(
    _tc3_body,
    grid=(GRID,),
    in_specs=[_pair_spec, _pair_spec, _d8_spec,
              _row_spec, _vecH_spec, _vecH_spec, _proj_spec, _vecD_spec,
              _proj_spec, _vecD_spec, _vecD_spec],
    out_specs=_row_spec,
    out_shape=jax.ShapeDtypeStruct((N, D), jnp.float32),
)


def kernel(x, t_emb, c_emb, edge_index, W1, b1, Wt1, bt1, Wc1, bc1, g1, beta1,
           W2, b2, Wt2, bt2, Wc2, bc2, g2, beta2):
    f32 = jnp.float32
    ones128 = jnp.ones((DEG_K, H), f32)
    z128 = jnp.zeros((OUT_K, H), f32)

    t2 = t_emb.reshape(1, -1)
    c2 = c_emb.reshape(1, -1)
    b1r, bt1r, bc1r = b1.reshape(1, -1), bt1.reshape(1, -1), bc1.reshape(1, -1)
    b2r, bt2r, bc2r = b2.reshape(1, -1), bt2.reshape(1, -1), bc2.reshape(1, -1)
    g1r, beta1r = g1.reshape(1, -1), beta1.reshape(1, -1)
    g2r, beta2r = g2.reshape(1, -1), beta2.reshape(1, -1)

    src = edge_index[0]
    dst = edge_index[1]
    srccat = jnp.concatenate([src, src + jnp.int32(N)])

    cnt = _deg_call(dst, z128, ones128)
    hsp, d8 = _tc1(x, cnt, g1r, beta1r, W1)
    agg = _agg_call(hsp.reshape(NC * N, H), srccat, dst, z128)
    hsp2 = _tc2(agg, hsp, d8, t2, c2, Wt1, bt1r, Wc1, bc1r, b1r,
                g2r, beta2r, W2)
    agg2 = _agg_call(hsp2.reshape(NC * N, H), srccat, dst, z128)
    y = _tc3(agg2, hsp2, d8, x, t2, c2, Wt2, bt2r, Wc2, bc2r, b2r)
    return y


# speedup vs baseline: 1.0957x; 1.0957x over previous
import jax
import jax.numpy as jnp
from jax import lax
from jax.experimental import pallas as pl
from jax.experimental.pallas import tpu as pltpu
from jax.experimental.pallas import tpu_sc as plsc

N = 10000
E = 160000
D = 256
H = 128
EPS = 1e-5

NC = 2
NS = 16

DEG_K = 40
DEG_EPT = E // (NC * NS)
DEG_NCH = DEG_EPT // DEG_K

AGG_K = 80
AGG_EPT = E // NS
AGG_NCH = AGG_EPT // AGG_K

OUT_K = 80
OUT_NCH = N // OUT_K
OUT_PER_TILE = -(-OUT_NCH // NS)

_MESH = plsc.VectorSubcoreMesh(core_axis_name="c", subcore_axis_name="s")


def _deg_body(dst, z128, ones128, cnt, didx, onesv, obuf, acc,
              sem_i, sem_s):
    c = lax.axis_index("c")
    s = lax.axis_index("s")
    base = (c * NS + s) * DEG_EPT

    def iload(i, carry):
        pltpu.async_copy(dst.at[pl.ds(base + i * DEG_K, DEG_K)],
                         didx.at[i], sem_i)
        return carry

    lax.fori_loop(0, DEG_NCH, iload, 0)

    pltpu.sync_copy(ones128, onesv)
    pltpu.sync_copy(z128, obuf)

    def zbody(j, carry):
        g = s + j * NS

        @pl.when(g < OUT_NCH)
        def _():
            pltpu.sync_copy(obuf, acc.at[pl.ds(g * OUT_K, OUT_K)])

        return carry

    lax.fori_loop(0, OUT_PER_TILE, zbody, 0)

    def idrain(i, carry):
        pltpu.make_async_copy(dst.at[pl.ds(base + i * DEG_K, DEG_K)],
                              didx.at[i], sem_i).wait()
        return carry

    lax.fori_loop(0, DEG_NCH, idrain, 0)
    plsc.subcore_barrier()

    def cbody(i, carry):
        pltpu.async_copy(onesv, acc.at[didx.at[i]], sem_s, add=True)
        return carry

    lax.fori_loop(0, DEG_NCH, cbody, 0)

    def cdrain(i, carry):
        pltpu.make_async_copy(onesv, acc.at[didx.at[i]], sem_s).wait()
        return carry

    lax.fori_loop(0, DEG_NCH, cdrain, 0)
    plsc.subcore_barrier()

    def obody(j, carry):
        g = s + j * NS

        @pl.when(g < OUT_NCH)
        def _():
            r = g * OUT_K
            pltpu.sync_copy(acc.at[pl.ds(r, OUT_K)], obuf)
            pltpu.sync_copy(obuf, cnt.at[c, pl.ds(r, OUT_K)])

        return carry

    lax.fori_loop(0, OUT_PER_TILE, obody, 0)


_deg_call = pl.kernel(
    _deg_body,
    out_type=jax.ShapeDtypeStruct((NC, N, H), jnp.float32),
    mesh=_MESH,
    scratch_types=[
        pltpu.VMEM((DEG_NCH, DEG_K), jnp.int32),
        pltpu.VMEM((DEG_K, H), jnp.float32),
        pltpu.VMEM((OUT_K, H), jnp.float32),
        pltpu.VMEM_SHARED((N, H), jnp.float32),
        pltpu.SemaphoreType.DMA,
        pltpu.SemaphoreType.DMA,
    ],
    name="sc_deg",
)


def _agg_body(hs2n, srccat, dst, z128, out, sidx, didx, gbuf, acc,
              sem_i, sem_g0, sem_g1):
    c = lax.axis_index("c")
    s = lax.axis_index("s")
    base = s * AGG_EPT

    pltpu.async_copy(srccat.at[pl.ds(c * E + base, AGG_EPT)], sidx, sem_i)

    def iload(i, carry):
        pltpu.async_copy(dst.at[pl.ds(base + i * AGG_K, AGG_K)],
                         didx.at[i], sem_i)
        return carry

    lax.fori_loop(0, AGG_NCH, iload, 0)

    pltpu.sync_copy(z128, gbuf.at[0])

    def zbody(j, carry):
        g = s + j * NS

        @pl.when(g < OUT_NCH)
        def _():
            pltpu.sync_copy(gbuf.at[0], acc.at[pl.ds(g * OUT_K, OUT_K)])

        return carry

    lax.fori_loop(0, OUT_PER_TILE, zbody, 0)

    pltpu.make_async_copy(srccat.at[pl.ds(c * E + base, AGG_EPT)],
                          sidx, sem_i).wait()

    def idrain(i, carry):
        pltpu.make_async_copy(dst.at[pl.ds(base + i * AGG_K, AGG_K)],
                              didx.at[i], sem_i).wait()
        return carry

    lax.fori_loop(0, AGG_NCH, idrain, 0)
    plsc.subcore_barrier()

    def gissue(k, buf, sem):
        pltpu.async_copy(hs2n.at[sidx.at[pl.ds(k * AGG_K, AGG_K)]],
                         gbuf.at[buf], sem)

    def gwait(k, buf, sem):
        pltpu.make_async_copy(hs2n.at[sidx.at[pl.ds(k * AGG_K, AGG_K)]],
                              gbuf.at[buf], sem).wait()

    def scat(k, buf):
        pltpu.sync_copy(gbuf.at[buf], acc.at[didx.at[k]], add=True)

    gissue(0, 0, sem_g0)

    def pair(i, carry):
        k = 2 * i
        gissue(k + 1, 1, sem_g1)
        gwait(k, 0, sem_g0)
        scat(k, 0)
        gissue(k + 2, 0, sem_g0)
        gwait(k + 1, 1, sem_g1)
        scat(k + 1, 1)
        return carry

    lax.fori_loop(0, (AGG_NCH - 1) // 2, pair, 0)
    gwait(AGG_NCH - 1, 0, sem_g0)
    scat(AGG_NCH - 1, 0)
    plsc.subcore_barrier()

    def obody(j, carry):
        g = s + j * NS

        @pl.when(g < OUT_NCH)
        def _():
            r = g * OUT_K
            pltpu.sync_copy(acc.at[pl.ds(r, OUT_K)], gbuf.at[0])
            pltpu.sync_copy(gbuf.at[0], out.at[c, pl.ds(r, OUT_K)])

        return carry

    lax.fori_loop(0, OUT_PER_TILE, obody, 0)


_agg_call = pl.kernel(
    _agg_body,
    out_type=jax.ShapeDtypeStruct((NC, N, H), jnp.float32),
    mesh=_MESH,
    scratch_types=[
        pltpu.VMEM((AGG_EPT,), jnp.int32),
        pltpu.VMEM((AGG_NCH, AGG_K), jnp.int32),
        pltpu.VMEM((2, AGG_K, H), jnp.float32),
        pltpu.VMEM_SHARED((N, H), jnp.float32),
        pltpu.SemaphoreType.DMA,
        pltpu.SemaphoreType.DMA,
        pltpu.SemaphoreType.DMA,
    ],
    name="sc_agg",
)


RB = 1000
GRID = N // RB


def _dinv_from_cnt(cnt_ref):
    deg = cnt_ref[0, :, 0:1] + cnt_ref[1, :, 0:1] + 1.0
    return lax.rsqrt(deg)


def _dinv_from_d8(d8_ref):
    return d8_ref[:, 0:1]


def _layer_norm_tc(xb, g, b):
    mu = jnp.mean(xb, axis=1, keepdims=True)
    xc = xb - mu
    var = jnp.mean(xc * xc, axis=1, keepdims=True)
    return xc * lax.rsqrt(var + EPS) * g + b


def _tc1_body(x_ref, cnt_ref, g_ref, b_ref, w_ref, hs_ref, d8_ref):
    ln = _layer_norm_tc(x_ref[...], g_ref[...], b_ref[...])
    h = jnp.dot(ln, w_ref[...], preferred_element_type=jnp.float32)
    dinv = _dinv_from_cnt(cnt_ref)
    hs = h * dinv
    hs_ref[0] = hs[:, :H]
    hs_ref[1] = hs[:, H:]
    d8_ref[...] = jnp.broadcast_to(dinv, (RB, 8))


_row_spec = pl.BlockSpec((RB, D), lambda i: (i, 0))
_pair_spec = pl.BlockSpec((NC, RB, H), lambda i: (0, i, 0))
_cnt_spec = pl.BlockSpec((NC, RB, H), lambda i: (0, i, 0))
_d8_spec = pl.BlockSpec((RB, 8), lambda i: (i, 0))
_vecD_spec = pl.BlockSpec((1, D), lambda i: (0, 0))
_vecH_spec = pl.BlockSpec((1, H), lambda i: (0, 0))
_mat_spec = pl.BlockSpec((D, D), lambda i: (0, 0))
_proj_spec = pl.BlockSpec((H, D), lambda i: (0, 0))

_tc1 = pl.pallas_call(
    _tc1_body,
    grid=(GRID,),
    in_specs=[_row_spec, _cnt_spec, _vecD_spec, _vecD_spec, _mat_spec],
    out_specs=[_pair_spec, _d8_spec],
    out_shape=[
        jax.ShapeDtypeStruct((NC, N, H), jnp.float32),
        jax.ShapeDtypeStruct((N, 8), jnp.float32),
    ],
)


def _tc2_body(a, hsp, d8, t, ce, wt, bt, wc, bc, b1, g2, be2, w2, o):
    dinv = _dinv_from_d8(d8)
    u0 = (a[0] + hsp[0]) * dinv
    u1 = (a[1] + hsp[1]) * dinv
    u = jnp.concatenate([u0, u1], axis=1)
    bias = (b1[...] + bt[...] + bc[...]
            + jnp.dot(t[...], wt[...], preferred_element_type=jnp.float32)
            + jnp.dot(ce[...], wc[...], preferred_element_type=jnp.float32))
    v = u + bias
    v = v * jax.nn.sigmoid(v)
    ln = _layer_norm_tc(v, g2[...], be2[...])
    h2 = jnp.dot(ln, w2[...], preferred_element_type=jnp.float32)
    hs2 = h2 * dinv
    o[0] = hs2[:, :H]
    o[1] = hs2[:, H:]


_tc2 = pl.pallas_call(
    _tc2_body,
    grid=(GRID,),
    in_specs=[_pair_spec, _pair_spec, _d8_spec,
              _vecH_spec, _vecH_spec, _proj_spec, _vecD_spec, _proj_spec,
              _vecD_spec, _vecD_spec, _vecD_spec, _vecD_spec, _mat_spec],
    out_specs=_pair_spec,
    out_shape=jax.ShapeDtypeStruct((NC, N, H), jnp.float32),
)


def _tc3_body(a, hsp, d8, x, t, ce, wt, bt, wc, bc, b2, o):
    dinv = _dinv_from_d8(d8)
    u0 = (a[0] + hsp[0]) * dinv
    u1 = (a[1] + hsp[1]) * dinv
    u = jnp.concatenate([u0, u1], axis=1)
    bias = (b2[...] + bt[...] + bc[...]
            + jnp.dot(t[...], wt[...], preferred_element_type=jnp.float32)
            + jnp.dot(ce[...], wc[...], preferred_element_type=jnp.float32))
    v = u + bias
    v = v * jax.nn.sigmoid(v)
    o[...] = v + x[...]


_tc3 = pl.pallas_call(
    _tc3_body,
    grid=(GRID,),
    in_specs=[_pair_spec, _pair_spec, _d8_spec,
              _row_spec, _vecH_spec, _vecH_spec, _proj_spec, _vecD_spec,
              _proj_spec, _vecD_spec, _vecD_spec],
    out_specs=_row_spec,
    out_shape=jax.ShapeDtypeStruct((N, D), jnp.float32),
)


def kernel(x, t_emb, c_emb, edge_index, W1, b1, Wt1, bt1, Wc1, bc1, g1, beta1,
           W2, b2, Wt2, bt2, Wc2, bc2, g2, beta2):
    f32 = jnp.float32
    ones128 = jnp.ones((DEG_K, H), f32)
    z128 = jnp.zeros((OUT_K, H), f32)

    t2 = t_emb.reshape(1, -1)
    c2 = c_emb.reshape(1, -1)
    b1r, bt1r, bc1r = b1.reshape(1, -1), bt1.reshape(1, -1), bc1.reshape(1, -1)
    b2r, bt2r, bc2r = b2.reshape(1, -1), bt2.reshape(1, -1), bc2.reshape(1, -1)
    g1r, beta1r = g1.reshape(1, -1), beta1.reshape(1, -1)
    g2r, beta2r = g2.reshape(1, -1), beta2.reshape(1, -1)

    src = edge_index[0]
    dst = edge_index[1]
    srccat = jnp.concatenate([src, src + jnp.int32(N)])

    cnt = _deg_call(dst, z128, ones128)
    hsp, d8 = _tc1(x, cnt, g1r, beta1r, W1)
    agg = _agg_call(hsp.reshape(NC * N, H), srccat, dst, z128)
    hsp2 = _tc2(agg, hsp, d8, t2, c2, Wt1, bt1r, Wc1, bc1r, b1r,
                g2r, beta2r, W2)
    agg2 = _agg_call(hsp2.reshape(NC * N, H), srccat, dst, z128)
    y = _tc3(agg2, hsp2, d8, x, t2, c2, Wt2, bt2r, Wc2, bc2r, b2r)
    return y
